# baseline (speedup 1.0000x reference)
import functools

import jax
import jax.numpy as jnp
from jax import lax
from jax.experimental import pallas as pl
from jax.experimental.pallas import tpu as pltpu
from jax.experimental.pallas import tpu_sc as plsc

T = 1024
H = 2048
I = 2048
E = 8
JITTER_EPS = 0.01

BT = 128
NA = 2 * T
NPAD = ((NA + E * (BT - 1) + BT - 1) // BT) * BT
NB = NPAD // BT
BI = 512
NI = I // BI
BP = 512
BH = 1024

NC = 2
NS = 16


@functools.lru_cache(maxsize=None)
def _mesh():
    return plsc.VectorSubcoreMesh(core_axis_name="c", subcore_axis_name="s")


def _routing_body(scores_ref, p1_ref, p2_ref, m1_ref, m2_ref, be_ref):
    scores = scores_ref[...]
    col = lax.broadcasted_iota(jnp.int32, scores.shape, 1)
    ninf = jnp.float32(-jnp.inf)

    m1 = jnp.max(scores, axis=1, keepdims=True)
    ind1 = jnp.min(jnp.where(scores == m1, col, E), axis=1, keepdims=True)
    factor1 = jnp.maximum(jnp.abs(scores), m1)
    mask1 = ((m1 - scores) / factor1) > (2.0 * JITTER_EPS)
    mg1 = jnp.where(mask1, ninf, scores)
    e1 = jnp.exp(mg1 - m1)
    p1 = e1 / jnp.sum(e1, axis=1, keepdims=True)
    mult1 = jnp.sum(jnp.where(col == ind1, p1, 0.0), axis=1, keepdims=True)

    masked_scores = jnp.where(col == ind1, ninf, scores)
    m2 = jnp.max(masked_scores, axis=1, keepdims=True)
    ind2 = jnp.min(jnp.where(masked_scores == m2, col, E), axis=1,
                   keepdims=True)
    factor2 = jnp.maximum(jnp.abs(scores), m2)
    mask2 = ((m2 - scores) / factor2) > (2.0 * JITTER_EPS)
    mg2 = jnp.where(mask2, ninf, masked_scores)
    e2 = jnp.exp(mg2 - m2)
    p2 = e2 / jnp.sum(e2, axis=1, keepdims=True)
    mult2 = jnp.sum(jnp.where(col == ind2, p2, 0.0), axis=1, keepdims=True)

    m1_ref[...] = mult1
    m2_ref[...] = mult2

    oh1 = (col == ind1)
    oh2 = (col == ind2)
    ohs = (oh1.astype(jnp.bfloat16) + oh2.astype(jnp.bfloat16))

    counts = jnp.sum(oh1.astype(jnp.float32) + oh2.astype(jnp.float32),
                     axis=0, keepdims=True).astype(jnp.int32)
    padded = ((counts + BT - 1) >> 7) << 7
    r8 = lax.broadcasted_iota(jnp.int32, (E, E), 0)
    c8 = lax.broadcasted_iota(jnp.int32, (E, E), 1)
    ends = lax.dot_general(
        padded.astype(jnp.bfloat16),
        (r8 <= c8).astype(jnp.bfloat16),
        (((1,), (0,)), ((), ())),
        preferred_element_type=jnp.float32).astype(jnp.int32)
    starts = ends - padded

    row = lax.broadcasted_iota(jnp.int32, (T, T), 0)
    colT = lax.broadcasted_iota(jnp.int32, (T, T), 1)
    ltri = (colT < row).astype(jnp.bfloat16)
    excl = lax.dot_general(ltri, ohs, (((1,), (0,)), ((), ())),
                           preferred_element_type=jnp.float32)
    excl = excl.astype(jnp.int32)

    sel_start1 = jnp.sum(jnp.where(col == ind1, starts, 0), axis=1,
                         keepdims=True)
    sel_start2 = jnp.sum(jnp.where(col == ind2, starts, 0), axis=1,
                         keepdims=True)
    rank1 = jnp.sum(jnp.where(col == ind1, excl, 0), axis=1, keepdims=True)
    rank2 = jnp.sum(jnp.where(col == ind2, excl, 0), axis=1, keepdims=True)
    p1_ref[...] = sel_start1 + rank1
    p2_ref[...] = sel_start2 + rank2

    posb = lax.broadcasted_iota(jnp.int32, (32, E), 0) * BT
    endsb = jnp.broadcast_to(ends, (32, E))
    be = jnp.sum((posb >= endsb).astype(jnp.int32), axis=1, keepdims=True)
    be_ref[...] = jnp.minimum(be, E - 1)


def _routing(scores):
    return pl.pallas_call(
        _routing_body,
        out_shape=[
            jax.ShapeDtypeStruct((T, 1), jnp.int32),
            jax.ShapeDtypeStruct((T, 1), jnp.int32),
            jax.ShapeDtypeStruct((T, 1), jnp.float32),
            jax.ShapeDtypeStruct((T, 1), jnp.float32),
            jax.ShapeDtypeStruct((32, 1), jnp.int32),
        ],
    )(scores)


def _build_body(p1_ref, p2_ref, m1_ref, m2_ref, tok_ref, wt_ref):
    s = pl.program_id(0)
    p1 = p1_ref[...]
    p2 = p2_ref[...]
    pblk = lax.broadcasted_iota(jnp.int32, (BP, T), 0) + s * BP
    tval = lax.broadcasted_iota(jnp.int32, (BP, T), 1)
    hit1 = pblk == p1
    hit2 = pblk == p2
    tok_ref[...] = (jnp.sum(jnp.where(hit1, tval, 0), axis=1, keepdims=True)
                    + jnp.sum(jnp.where(hit2, tval, 0), axis=1,
                              keepdims=True))
    wt_ref[...] = (jnp.sum(jnp.where(hit1, m1_ref[...], 0.0), axis=1,
                           keepdims=True)
                   + jnp.sum(jnp.where(hit2, m2_ref[...], 0.0), axis=1,
                             keepdims=True))


def _build(p1, p2, m1, m2):
    return pl.pallas_call(
        _build_body,
        grid=(NPAD // BP,),
        in_specs=[
            pl.BlockSpec((1, T), lambda s: (0, 0)),
            pl.BlockSpec((1, T), lambda s: (0, 0)),
            pl.BlockSpec((1, T), lambda s: (0, 0)),
            pl.BlockSpec((1, T), lambda s: (0, 0)),
        ],
        out_specs=[
            pl.BlockSpec((BP, 1), lambda s: (s, 0)),
            pl.BlockSpec((BP, 1), lambda s: (s, 0)),
        ],
        out_shape=[
            jax.ShapeDtypeStruct((NPAD, 1), jnp.int32),
            jax.ShapeDtypeStruct((NPAD, 1), jnp.float32),
        ],
        compiler_params=pltpu.CompilerParams(
            dimension_semantics=("arbitrary",)),
    )(p1, p2, m1, m2)


_ROWS_W = NPAD // (NC * NS)
_GCH = 32


@functools.lru_cache(maxsize=None)
def _gather_kernel():
    return pl.kernel(
        _gather_body, mesh=_mesh(),
        out_type=jax.ShapeDtypeStruct((NPAD, H), jnp.float32),
        scratch_types=[
            pltpu.VMEM((_GCH,), jnp.int32),
            pltpu.VMEM((_GCH, H), jnp.float32),
            pltpu.SemaphoreType.DMA,
        ],
    )


def _gather(x, tok):
    return _gather_kernel()(x, tok)


def _gather_body(x_hbm, tok_hbm, xg_hbm, idxv, rowsv, sem):
    wid = lax.axis_index("s") * NC + lax.axis_index("c")
    for c in range(_ROWS_W // _GCH):
        base = wid * _ROWS_W + c * _GCH
        pltpu.sync_copy(tok_hbm.at[pl.ds(base, _GCH)], idxv)
        pltpu.async_copy(x_hbm.at[idxv], rowsv, sem).wait()
        pltpu.sync_copy(rowsv, xg_hbm.at[pl.ds(base, _GCH)])


def _ffn_a_body(be_ref, xg_ref, w1_ref, w3_ref, act_ref):
    x = xg_ref[...]
    h1 = lax.dot_general(x, w1_ref[0], (((1,), (1,)), ((), ())),
                         preferred_element_type=jnp.float32)
    h3 = lax.dot_general(x, w3_ref[0], (((1,), (1,)), ((), ())),
                         preferred_element_type=jnp.float32)
    act_ref[...] = h1 * jax.nn.sigmoid(h1) * h3


def _ffn_a(be, xg, ws):
    spec = pltpu.PrefetchScalarGridSpec(
        num_scalar_prefetch=1,
        grid=(NI, NB),
        in_specs=[
            pl.BlockSpec((BT, H), lambda i, b, be: (b, 0)),
            pl.BlockSpec((1, BI, H), lambda i, b, be: (be[b], i, 0)),
            pl.BlockSpec((1, BI, H), lambda i, b, be: (be[b], i + NI, 0)),
        ],
        out_specs=pl.BlockSpec((BT, BI), lambda i, b, be: (b, i)),
    )
    return pl.pallas_call(
        _ffn_a_body,
        grid_spec=spec,
        out_shape=jax.ShapeDtypeStruct((NPAD, I), jnp.float32),
        compiler_params=pltpu.CompilerParams(
            dimension_semantics=("arbitrary", "arbitrary")),
    )(be, xg, ws, ws)


def _ffn_b_body(be_ref, act_ref, w2_ref, wt_ref, yg_ref):
    a = act_ref[...]
    y = lax.dot_general(a, w2_ref[0], (((1,), (1,)), ((), ())),
                        preferred_element_type=jnp.float32)
    yg_ref[...] = y * wt_ref[...]


def _ffn_b(be, act, w2s, wtcol):
    spec = pltpu.PrefetchScalarGridSpec(
        num_scalar_prefetch=1,
        grid=(H // BH, NB),
        in_specs=[
            pl.BlockSpec((BT, I), lambda h, b, be: (b, 0)),
            pl.BlockSpec((1, BH, I), lambda h, b, be: (be[b], h, 0)),
            pl.BlockSpec((BT, 1), lambda h, b, be: (b, 0)),
        ],
        out_specs=pl.BlockSpec((BT, BH), lambda h, b, be: (b, h)),
    )
    return pl.pallas_call(
        _ffn_b_body,
        grid_spec=spec,
        out_shape=jax.ShapeDtypeStruct((NPAD, H), jnp.float32),
        compiler_params=pltpu.CompilerParams(
            dimension_semantics=("arbitrary", "arbitrary")),
    )(be, act, w2s, wtcol)


_TOK_W = T // (NC * NS)
_TCH = 16


@functools.lru_cache(maxsize=None)
def _combine_kernel():
    return pl.kernel(
        _combine_body, mesh=_mesh(),
        out_type=jax.ShapeDtypeStruct((T, H), jnp.float32),
        scratch_types=[
            pltpu.VMEM((2 * _TCH,), jnp.int32),
            pltpu.VMEM((2 * _TCH, H), jnp.float32),
            pltpu.VMEM((_TCH, H), jnp.float32),
            pltpu.SemaphoreType.DMA,
        ],
    )


def _combine(yg, inv):
    return _combine_kernel()(yg, inv)


def _combine_body(yg_hbm, inv_hbm, out_hbm, idxv, pairv, outv, sem):
    wid = lax.axis_index("s") * NC + lax.axis_index("c")
    for sub in range(_TOK_W // _TCH):
        tbase = wid * _TOK_W + sub * _TCH
        pltpu.sync_copy(inv_hbm.at[pl.ds(tbase * 2, 2 * _TCH)], idxv)
        pltpu.async_copy(yg_hbm.at[idxv], pairv, sem).wait()

        def add_body(s, carry):
            for j in range(_TCH):
                outv[j, pl.ds(s * 16, 16)] = (
                    pairv[2 * j, pl.ds(s * 16, 16)]
                    + pairv[2 * j + 1, pl.ds(s * 16, 16)])
            return carry
        lax.fori_loop(0, H // 16, add_body, 0)
        pltpu.sync_copy(outv, out_hbm.at[pl.ds(tbase, _TCH)])


@jax.jit
def kernel(hidden_states, gate_weight, ws, w2s):
    x = hidden_states
    scores = x @ gate_weight.T
    pos1, pos2, m1, m2, be = _routing(scores)
    tok, wt_sorted = _build(pos1.reshape(1, T), pos2.reshape(1, T),
                            m1.reshape(1, T), m2.reshape(1, T))
    inv = jnp.concatenate([pos1, pos2], axis=1).reshape(NA)
    xg = _gather(x, tok.reshape(NPAD))
    act = _ffn_a(be.reshape(32), xg, ws)
    yg = _ffn_b(be.reshape(32), act, w2s, wt_sorted)
    return _combine(yg, inv)


# speedup vs baseline: 1.1601x; 1.1601x over previous
import functools

import jax
import jax.numpy as jnp
from jax import lax
from jax.experimental import pallas as pl
from jax.experimental.pallas import tpu as pltpu
from jax.experimental.pallas import tpu_sc as plsc

T = 1024
H = 2048
I = 2048
E = 8
JITTER_EPS = 0.01

BT = 128
NA = 2 * T
NPAD = ((NA + E * (BT - 1) + BT - 1) // BT) * BT
NB = NPAD // BT
BI = 512
NI = I // BI
BP = 512
BH = 1024

NC = 2
NS = 16


@functools.lru_cache(maxsize=None)
def _mesh():
    return plsc.VectorSubcoreMesh(core_axis_name="c", subcore_axis_name="s")


def _routing_body(scores_ref, p1_ref, p2_ref, m1_ref, m2_ref, be_ref):
    scores = scores_ref[...]
    col = lax.broadcasted_iota(jnp.int32, scores.shape, 1)
    ninf = jnp.float32(-jnp.inf)

    m1 = jnp.max(scores, axis=1, keepdims=True)
    ind1 = jnp.min(jnp.where(scores == m1, col, E), axis=1, keepdims=True)
    factor1 = jnp.maximum(jnp.abs(scores), m1)
    mask1 = ((m1 - scores) / factor1) > (2.0 * JITTER_EPS)
    mg1 = jnp.where(mask1, ninf, scores)
    e1 = jnp.exp(mg1 - m1)
    p1 = e1 / jnp.sum(e1, axis=1, keepdims=True)
    mult1 = jnp.sum(jnp.where(col == ind1, p1, 0.0), axis=1, keepdims=True)

    masked_scores = jnp.where(col == ind1, ninf, scores)
    m2 = jnp.max(masked_scores, axis=1, keepdims=True)
    ind2 = jnp.min(jnp.where(masked_scores == m2, col, E), axis=1,
                   keepdims=True)
    factor2 = jnp.maximum(jnp.abs(scores), m2)
    mask2 = ((m2 - scores) / factor2) > (2.0 * JITTER_EPS)
    mg2 = jnp.where(mask2, ninf, masked_scores)
    e2 = jnp.exp(mg2 - m2)
    p2 = e2 / jnp.sum(e2, axis=1, keepdims=True)
    mult2 = jnp.sum(jnp.where(col == ind2, p2, 0.0), axis=1, keepdims=True)

    m1_ref[...] = mult1
    m2_ref[...] = mult2

    oh1 = (col == ind1)
    oh2 = (col == ind2)
    ohs = (oh1.astype(jnp.bfloat16) + oh2.astype(jnp.bfloat16))

    counts = jnp.sum(oh1.astype(jnp.float32) + oh2.astype(jnp.float32),
                     axis=0, keepdims=True).astype(jnp.int32)
    padded = ((counts + BT - 1) >> 7) << 7
    r8 = lax.broadcasted_iota(jnp.int32, (E, E), 0)
    c8 = lax.broadcasted_iota(jnp.int32, (E, E), 1)
    ends = lax.dot_general(
        padded.astype(jnp.bfloat16),
        (r8 <= c8).astype(jnp.bfloat16),
        (((1,), (0,)), ((), ())),
        preferred_element_type=jnp.float32).astype(jnp.int32)
    starts = ends - padded

    row = lax.broadcasted_iota(jnp.int32, (T, T), 0)
    colT = lax.broadcasted_iota(jnp.int32, (T, T), 1)
    ltri = (colT < row).astype(jnp.bfloat16)
    excl = lax.dot_general(ltri, ohs, (((1,), (0,)), ((), ())),
                           preferred_element_type=jnp.float32)
    excl = excl.astype(jnp.int32)

    sel_start1 = jnp.sum(jnp.where(col == ind1, starts, 0), axis=1,
                         keepdims=True)
    sel_start2 = jnp.sum(jnp.where(col == ind2, starts, 0), axis=1,
                         keepdims=True)
    rank1 = jnp.sum(jnp.where(col == ind1, excl, 0), axis=1, keepdims=True)
    rank2 = jnp.sum(jnp.where(col == ind2, excl, 0), axis=1, keepdims=True)
    p1_ref[...] = sel_start1 + rank1
    p2_ref[...] = sel_start2 + rank2

    posb = lax.broadcasted_iota(jnp.int32, (32, E), 0) * BT
    endsb = jnp.broadcast_to(ends, (32, E))
    be = jnp.sum((posb >= endsb).astype(jnp.int32), axis=1, keepdims=True)
    be_ref[...] = jnp.minimum(be, E - 1)


def _routing(scores):
    return pl.pallas_call(
        _routing_body,
        out_shape=[
            jax.ShapeDtypeStruct((T, 1), jnp.int32),
            jax.ShapeDtypeStruct((T, 1), jnp.int32),
            jax.ShapeDtypeStruct((T, 1), jnp.float32),
            jax.ShapeDtypeStruct((T, 1), jnp.float32),
            jax.ShapeDtypeStruct((32, 1), jnp.int32),
        ],
    )(scores)


def _build_body(p1_ref, p2_ref, m1_ref, m2_ref, x_ref, xg_ref, wt_ref):
    s = pl.program_id(0)
    p1 = p1_ref[...]
    p2 = p2_ref[...]
    pblk = lax.broadcasted_iota(jnp.int32, (BP, T), 0) + s * BP
    hit1 = pblk == p1
    hit2 = pblk == p2
    perm = hit1.astype(jnp.bfloat16) + hit2.astype(jnp.bfloat16)
    xg_ref[...] = lax.dot_general(perm, x_ref[...],
                                  (((1,), (0,)), ((), ())),
                                  preferred_element_type=jnp.float32)
    wt_ref[...] = (jnp.sum(jnp.where(hit1, m1_ref[...], 0.0), axis=1,
                           keepdims=True)
                   + jnp.sum(jnp.where(hit2, m2_ref[...], 0.0), axis=1,
                             keepdims=True))


def _build(p1, p2, m1, m2, x):
    return pl.pallas_call(
        _build_body,
        grid=(NPAD // BP,),
        in_specs=[
            pl.BlockSpec((1, T), lambda s: (0, 0)),
            pl.BlockSpec((1, T), lambda s: (0, 0)),
            pl.BlockSpec((1, T), lambda s: (0, 0)),
            pl.BlockSpec((1, T), lambda s: (0, 0)),
            pl.BlockSpec((T, H), lambda s: (0, 0)),
        ],
        out_specs=[
            pl.BlockSpec((BP, H), lambda s: (s, 0)),
            pl.BlockSpec((BP, 1), lambda s: (s, 0)),
        ],
        out_shape=[
            jax.ShapeDtypeStruct((NPAD, H), jnp.float32),
            jax.ShapeDtypeStruct((NPAD, 1), jnp.float32),
        ],
        compiler_params=pltpu.CompilerParams(
            dimension_semantics=("arbitrary",)),
    )(p1, p2, m1, m2, x)


_ROWS_W = NPAD // (NC * NS)
_GCH = 32


@functools.lru_cache(maxsize=None)
def _gather_kernel():
    return pl.kernel(
        _gather_body, mesh=_mesh(),
        out_type=jax.ShapeDtypeStruct((NPAD, H), jnp.float32),
        scratch_types=[
            pltpu.VMEM((_GCH,), jnp.int32),
            pltpu.VMEM((_GCH, H), jnp.float32),
            pltpu.SemaphoreType.DMA,
        ],
    )


def _gather(x, tok):
    return _gather_kernel()(x, tok)


def _gather_body(x_hbm, tok_hbm, xg_hbm, idxv, rowsv, sem):
    wid = lax.axis_index("s") * NC + lax.axis_index("c")
    for c in range(_ROWS_W // _GCH):
        base = wid * _ROWS_W + c * _GCH
        pltpu.sync_copy(tok_hbm.at[pl.ds(base, _GCH)], idxv)
        pltpu.async_copy(x_hbm.at[idxv], rowsv, sem).wait()
        pltpu.sync_copy(rowsv, xg_hbm.at[pl.ds(base, _GCH)])


def _ffn_a_body(be_ref, xg_ref, w1_ref, w3_ref, act_ref):
    x = xg_ref[...]
    h1 = lax.dot_general(x, w1_ref[0], (((1,), (1,)), ((), ())),
                         preferred_element_type=jnp.float32)
    h3 = lax.dot_general(x, w3_ref[0], (((1,), (1,)), ((), ())),
                         preferred_element_type=jnp.float32)
    act_ref[...] = h1 * jax.nn.sigmoid(h1) * h3


def _ffn_a(be, xg, ws):
    spec = pltpu.PrefetchScalarGridSpec(
        num_scalar_prefetch=1,
        grid=(NI, NB),
        in_specs=[
            pl.BlockSpec((BT, H), lambda i, b, be: (b, 0)),
            pl.BlockSpec((1, BI, H), lambda i, b, be: (be[b], i, 0)),
            pl.BlockSpec((1, BI, H), lambda i, b, be: (be[b], i + NI, 0)),
        ],
        out_specs=pl.BlockSpec((BT, BI), lambda i, b, be: (b, i)),
    )
    return pl.pallas_call(
        _ffn_a_body,
        grid_spec=spec,
        out_shape=jax.ShapeDtypeStruct((NPAD, I), jnp.float32),
        compiler_params=pltpu.CompilerParams(
            dimension_semantics=("arbitrary", "arbitrary")),
    )(be, xg, ws, ws)


def _ffn_b_body(be_ref, act_ref, w2_ref, wt_ref, yg_ref):
    a = act_ref[...]
    y = lax.dot_general(a, w2_ref[0], (((1,), (1,)), ((), ())),
                        preferred_element_type=jnp.float32)
    yg_ref[...] = y * wt_ref[...]


def _ffn_b(be, act, w2s, wtcol):
    spec = pltpu.PrefetchScalarGridSpec(
        num_scalar_prefetch=1,
        grid=(H // BH, NB),
        in_specs=[
            pl.BlockSpec((BT, I), lambda h, b, be: (b, 0)),
            pl.BlockSpec((1, BH, I), lambda h, b, be: (be[b], h, 0)),
            pl.BlockSpec((BT, 1), lambda h, b, be: (b, 0)),
        ],
        out_specs=pl.BlockSpec((BT, BH), lambda h, b, be: (b, h)),
    )
    return pl.pallas_call(
        _ffn_b_body,
        grid_spec=spec,
        out_shape=jax.ShapeDtypeStruct((NPAD, H), jnp.float32),
        compiler_params=pltpu.CompilerParams(
            dimension_semantics=("arbitrary", "arbitrary")),
    )(be, act, w2s, wtcol)


_TOK_W = T // (NC * NS)
_TCH = 16


@functools.lru_cache(maxsize=None)
def _combine_kernel():
    return pl.kernel(
        _combine_body, mesh=_mesh(),
        out_type=jax.ShapeDtypeStruct((T, H), jnp.float32),
        scratch_types=[
            pltpu.VMEM((2 * _TCH,), jnp.int32),
            pltpu.VMEM((2 * _TCH, H), jnp.float32),
            pltpu.VMEM((_TCH, H), jnp.float32),
            pltpu.SemaphoreType.DMA,
        ],
    )


def _combine(yg, inv):
    return _combine_kernel()(yg, inv)


def _combine_body(yg_hbm, inv_hbm, out_hbm, idxv, pairv, outv, sem):
    wid = lax.axis_index("s") * NC + lax.axis_index("c")
    for sub in range(_TOK_W // _TCH):
        tbase = wid * _TOK_W + sub * _TCH
        pltpu.sync_copy(inv_hbm.at[pl.ds(tbase * 2, 2 * _TCH)], idxv)
        pltpu.async_copy(yg_hbm.at[idxv], pairv, sem).wait()

        def add_body(s, carry):
            for j in range(_TCH):
                outv[j, pl.ds(s * 16, 16)] = (
                    pairv[2 * j, pl.ds(s * 16, 16)]
                    + pairv[2 * j + 1, pl.ds(s * 16, 16)])
            return carry
        lax.fori_loop(0, H // 16, add_body, 0)
        pltpu.sync_copy(outv, out_hbm.at[pl.ds(tbase, _TCH)])


@jax.jit
def kernel(hidden_states, gate_weight, ws, w2s):
    x = hidden_states
    scores = x @ gate_weight.T
    pos1, pos2, m1, m2, be = _routing(scores)
    xg, wt_sorted = _build(pos1.reshape(1, T), pos2.reshape(1, T),
                           m1.reshape(1, T), m2.reshape(1, T), x)
    inv = jnp.concatenate([pos1, pos2], axis=1).reshape(NA)
    act = _ffn_a(be.reshape(32), xg, ws)
    yg = _ffn_b(be.reshape(32), act, w2s, wt_sorted)
    return _combine(yg, inv)


# speedup vs baseline: 2.0827x; 1.7952x over previous
import functools

import jax
import jax.numpy as jnp
from jax.experimental import pallas as pl
from jax.experimental.pallas import tpu as pltpu

T = 1024
H = 2048
I = 2048
E = 8
JITTER_EPS = 0.01
BI = 512
NI = I // BI


def _routing_body(scores_ref, comb_ref):
    scores = scores_ref[...]
    col = jax.lax.broadcasted_iota(jnp.int32, scores.shape, 1)
    ninf = jnp.float32(-jnp.inf)

    m1 = jnp.max(scores, axis=1, keepdims=True)
    ind1 = jnp.min(jnp.where(scores == m1, col, E), axis=1, keepdims=True)
    factor1 = jnp.maximum(jnp.abs(scores), m1)
    mask1 = ((m1 - scores) / factor1) > (2.0 * JITTER_EPS)
    mg1 = jnp.where(mask1, ninf, scores)
    e1 = jnp.exp(mg1 - m1)
    p1 = e1 / jnp.sum(e1, axis=1, keepdims=True)
    mult1 = jnp.sum(jnp.where(col == ind1, p1, 0.0), axis=1, keepdims=True)

    masked_scores = jnp.where(col == ind1, ninf, scores)
    m2 = jnp.max(masked_scores, axis=1, keepdims=True)
    ind2 = jnp.min(jnp.where(masked_scores == m2, col, E), axis=1, keepdims=True)
    factor2 = jnp.maximum(jnp.abs(scores), m2)
    mask2 = ((m2 - scores) / factor2) > (2.0 * JITTER_EPS)
    mg2 = jnp.where(mask2, ninf, masked_scores)
    e2 = jnp.exp(mg2 - m2)
    p2 = e2 / jnp.sum(e2, axis=1, keepdims=True)
    mult2 = jnp.sum(jnp.where(col == ind2, p2, 0.0), axis=1, keepdims=True)

    comb_ref[...] = (jnp.where(col == ind1, mult1, 0.0)
                     + jnp.where(col == ind2, mult2, 0.0))


def _routing(scores):
    return pl.pallas_call(
        _routing_body,
        out_shape=jax.ShapeDtypeStruct((T, E), jnp.float32),
    )(scores)


def _ffn_body(x_ref, w1_ref, w3_ref, w2_ref, comb_ref, out_ref):
    e = pl.program_id(0)
    i = pl.program_id(1)

    @pl.when((e == 0) & (i == 0))
    def _init():
        out_ref[...] = jnp.zeros_like(out_ref)

    x = x_ref[...]
    w1 = w1_ref[0]
    w3 = w3_ref[0]
    h1 = jax.lax.dot_general(x, w1, (((1,), (1,)), ((), ())),
                             preferred_element_type=jnp.float32)
    h3 = jax.lax.dot_general(x, w3, (((1,), (1,)), ((), ())),
                             preferred_element_type=jnp.float32)
    act = h1 * jax.nn.sigmoid(h1) * h3
    w2 = w2_ref[0]
    y = jax.lax.dot_general(act, w2, (((1,), (1,)), ((), ())),
                            preferred_element_type=jnp.float32)
    comb = comb_ref[...]
    ecol = jax.lax.broadcasted_iota(jnp.int32, comb.shape, 1)
    cvec = jnp.sum(jnp.where(ecol == e, comb, 0.0), axis=1, keepdims=True)
    out_ref[...] += cvec * y


def _ffn(x_bf16, ws, w2s, comb):
    grid = (E, NI)
    return pl.pallas_call(
        _ffn_body,
        grid=grid,
        in_specs=[
            pl.BlockSpec((T, H), lambda e, i: (0, 0)),
            pl.BlockSpec((1, BI, H), lambda e, i: (e, i, 0)),
            pl.BlockSpec((1, BI, H), lambda e, i: (e, i + NI, 0)),
            pl.BlockSpec((1, H, BI), lambda e, i: (e, 0, i)),
            pl.BlockSpec((T, E), lambda e, i: (0, 0)),
        ],
        out_specs=pl.BlockSpec((T, H), lambda e, i: (0, 0)),
        out_shape=jax.ShapeDtypeStruct((T, H), jnp.float32),
        compiler_params=pltpu.CompilerParams(
            dimension_semantics=("arbitrary", "arbitrary")),
    )(x_bf16, ws, ws, w2s, comb)


@jax.jit
def kernel(hidden_states, gate_weight, ws, w2s):
    scores = hidden_states @ gate_weight.T
    comb = _routing(scores)
    return _ffn(hidden_states, ws, w2s, comb)
